# baseline (speedup 1.0000x reference)
import functools

import jax
import jax.numpy as jnp
from jax import lax
from jax.experimental import pallas as pl
from jax.experimental.pallas import tpu as pltpu
from jax.experimental.pallas import tpu_sc as plsc

NC = 2
NS = 16
NW = NC * NS
SL = 128


def _do_part(table_hbm, idx_hbm, out_hbm, idx_v, rows_v, sem, wid,
             n_slices_w, group):
    slice_base = wid * n_slices_w
    n_groups = n_slices_w // group

    @pl.loop(0, n_groups)
    def _(g):
        s0 = slice_base + g * group
        pltpu.sync_copy(idx_hbm.at[pl.ds(s0 * SL, group * SL)],
                        idx_v.at[pl.ds(0, group * SL)])
        copies = [
            pltpu.async_copy(table_hbm.at[idx_v.at[pl.ds(j * SL, SL)]],
                             rows_v.at[pl.ds(j * SL, SL)], sem)
            for j in range(group)
        ]
        for c in copies:
            c.wait()
        pltpu.sync_copy(rows_v.at[pl.ds(0, group * SL)],
                        out_hbm.at[pl.ds(s0 * SL, group * SL)])


@functools.cache
def _build(n_ctx, n_q, vocab, dim):
    ctx_slices_w = n_ctx // SL // NW
    q_slices_w = n_q // SL // NW
    assert ctx_slices_w * SL * NW == n_ctx and q_slices_w * SL * NW == n_q
    g_ctx = 10
    g_q = q_slices_w
    g_max = max(g_ctx, g_q)
    mesh = plsc.VectorSubcoreMesh(core_axis_name="c", subcore_axis_name="s")

    @functools.partial(
        pl.kernel,
        out_type=(
            jax.ShapeDtypeStruct((n_ctx, dim), jnp.float32),
            jax.ShapeDtypeStruct((n_q, dim), jnp.float32),
        ),
        mesh=mesh,
        scratch_types=[
            pltpu.VMEM((g_max * SL,), jnp.int32),
            pltpu.VMEM((g_max * SL, dim), jnp.float32),
            pltpu.SemaphoreType.DMA,
        ],
        compiler_params=pltpu.CompilerParams(use_tc_tiling_on_sc=False),
    )
    def emb_gather(ctx_idx, q_idx, table, out_ctx, out_q, idx_v, rows_v, sem):
        wid = lax.axis_index("s") * NC + lax.axis_index("c")
        _do_part(table, ctx_idx, out_ctx, idx_v, rows_v, sem, wid,
                 ctx_slices_w, g_ctx)
        _do_part(table, q_idx, out_q, idx_v, rows_v, sem, wid,
                 q_slices_w, g_q)

    return emb_gather


def kernel(input_context, input_question, word_embeddings):
    b, l_ctx = input_context.shape
    _, l_q = input_question.shape
    vocab, dim = word_embeddings.shape
    n_ctx = b * l_ctx
    n_q = b * l_q
    fn = _build(n_ctx, n_q, vocab, dim)
    out_ctx, out_q = fn(
        input_context.reshape(n_ctx),
        input_question.reshape(n_q),
        word_embeddings,
    )
    return (out_ctx.reshape(b, l_ctx, dim), out_q.reshape(b, l_q, dim))


# speedup vs baseline: 1.0027x; 1.0027x over previous
import functools

import jax
import jax.numpy as jnp
from jax import lax
from jax.experimental import pallas as pl
from jax.experimental.pallas import tpu as pltpu
from jax.experimental.pallas import tpu_sc as plsc

NC = 2
NS = 16
NW = NC * NS
SL = 128


@functools.cache
def _build(n_ctx, n_q, vocab, dim):
    ctx_sl = n_ctx // SL // NW
    q_sl = n_q // SL // NW
    assert ctx_sl * SL * NW == n_ctx and q_sl * SL * NW == n_q
    g_ctx = 10
    n_groups = ctx_sl // g_ctx
    assert n_groups * g_ctx == ctx_sl and q_sl <= g_ctx
    mesh = plsc.VectorSubcoreMesh(core_axis_name="c", subcore_axis_name="s")

    @functools.partial(
        pl.kernel,
        out_type=(
            jax.ShapeDtypeStruct((n_ctx, dim), jnp.float32),
            jax.ShapeDtypeStruct((n_q, dim), jnp.float32),
        ),
        mesh=mesh,
        scratch_types=[
            pltpu.VMEM(((ctx_sl + q_sl) * SL,), jnp.int32),
            pltpu.VMEM((2 * g_ctx * SL, dim), jnp.float32),
            pltpu.SemaphoreType.DMA,
            pltpu.SemaphoreType.DMA,
        ],
        compiler_params=pltpu.CompilerParams(use_tc_tiling_on_sc=False),
    )
    def emb_gather(ctx_idx, q_idx, table, out_ctx, out_q, idx_v, rows_v,
                   gsem, osem):
        wid = lax.axis_index("s") * NC + lax.axis_index("c")

        pltpu.sync_copy(ctx_idx.at[pl.ds(wid * ctx_sl * SL, ctx_sl * SL)],
                        idx_v.at[pl.ds(0, ctx_sl * SL)])
        pltpu.sync_copy(q_idx.at[pl.ds(wid * q_sl * SL, q_sl * SL)],
                        idx_v.at[pl.ds(ctx_sl * SL, q_sl * SL)])

        work = [(out_ctx, g * g_ctx * SL, wid * ctx_sl * SL + g * g_ctx * SL,
                 g_ctx) for g in range(n_groups)]
        work.append((out_q, ctx_sl * SL, wid * q_sl * SL, q_sl))

        pending = [None, None]
        for g, (out, idx_base, out_base, n_slices) in enumerate(work):
            rows_lo = (g % 2) * g_ctx * SL
            if pending[g % 2] is not None:
                pending[g % 2].wait()
            gathers = [
                pltpu.async_copy(
                    table.at[idx_v.at[pl.ds(idx_base + j * SL, SL)]],
                    rows_v.at[pl.ds(rows_lo + j * SL, SL)], gsem)
                for j in range(n_slices)
            ]
            for gat in gathers:
                gat.wait()
            pending[g % 2] = pltpu.async_copy(
                rows_v.at[pl.ds(rows_lo, n_slices * SL)],
                out.at[pl.ds(out_base, n_slices * SL)], osem)
        for p in pending:
            if p is not None:
                p.wait()

    return emb_gather


def kernel(input_context, input_question, word_embeddings):
    b, l_ctx = input_context.shape
    _, l_q = input_question.shape
    vocab, dim = word_embeddings.shape
    n_ctx = b * l_ctx
    n_q = b * l_q
    fn = _build(n_ctx, n_q, vocab, dim)
    out_ctx, out_q = fn(
        input_context.reshape(n_ctx),
        input_question.reshape(n_q),
        word_embeddings,
    )
    return (out_ctx.reshape(b, l_ctx, dim), out_q.reshape(b, l_q, dim))
